# Initial kernel scaffold; baseline (speedup 1.0000x reference)
import jax
import jax.numpy as jnp
from jax.experimental import pallas as pl


def kernel(x, edge_index, W1, b1, W2, b2, W3, b3):
    raise NotImplementedError("write your pallas kernel here")


# speedup vs baseline: 6.0743x; 6.0743x over previous
import functools

import jax
import jax.numpy as jnp
from jax import lax
from jax.experimental import pallas as pl
from jax.experimental.pallas import tpu as pltpu
from jax.experimental.pallas import tpu_sc as plsc

N = 10000
E = 320000
D = 128
NP = 10240

NC = 2
NS = 16
NW = NC * NS
CH = 128
NCH = 80
EP = NW * NCH * CH
EPW = EP // NW
RPT = NP // NS
DZC = NP // 8

_mesh = plsc.VectorSubcoreMesh(
    core_axis_name="c", subcore_axis_name="s", num_cores=NC, num_subcores=NS
)


def _deg_body(dst_hbm, out_hbm, dst_v, ones_v, zeros_v, deg_acc):
    c = lax.axis_index("c")
    s = lax.axis_index("s")
    wid = s * NC + c

    @pl.when(s < 8)
    def _():
        def zfill(i, _):
            zeros_v[pl.ds(i * 16, 16)] = jnp.zeros((16,), jnp.float32)
            return _
        lax.fori_loop(0, DZC // 16, zfill, None)
        pltpu.sync_copy(zeros_v, deg_acc.at[pl.ds(s * DZC, DZC)])

    def ofill(i, _):
        ones_v[pl.ds(i * 16, 16)] = jnp.ones((16,), jnp.float32)
        return _
    lax.fori_loop(0, 128 // 16, ofill, None)

    pltpu.sync_copy(dst_hbm.at[wid], dst_v)
    plsc.subcore_barrier()

    def body(j, _):
        pltpu.sync_copy(ones_v, deg_acc.at[dst_v.at[j]], add=True)
        return _
    lax.fori_loop(0, NCH, body, None)

    plsc.subcore_barrier()

    @pl.when(s == 0)
    def _():
        pltpu.sync_copy(deg_acc, out_hbm.at[pl.ds(c * NP, NP)])


_deg_kernel = functools.partial(
    pl.kernel,
    out_type=jax.ShapeDtypeStruct((NC * NP,), jnp.float32),
    mesh=_mesh,
    scratch_types=[
        pltpu.VMEM((NCH, CH), jnp.int32),
        pltpu.VMEM((128,), jnp.float32),
        pltpu.VMEM((DZC,), jnp.float32),
        pltpu.VMEM_SHARED((NP,), jnp.float32),
    ],
)(_deg_body)


def _agg_body(u_hbm, src_hbm, dst_hbm, out_hbm,
              src_v, dst_v, rows_v, acc, sem):
    c = lax.axis_index("c")
    s = lax.axis_index("s")
    wid = s * NC + c

    def zfill16(i, _):
        r = i // (D // 16)
        q = i % (D // 16)
        rows_v[r, pl.ds(q * 16, 16)] = jnp.zeros((16,), jnp.float32)
        return _
    lax.fori_loop(0, CH * (D // 16), zfill16, None)

    def zcopy(k, _):
        pltpu.sync_copy(rows_v, acc.at[pl.ds(s * RPT + k * CH, CH)])
        return _
    lax.fori_loop(0, RPT // CH, zcopy, None)

    pltpu.sync_copy(src_hbm.at[wid], src_v)
    pltpu.sync_copy(dst_hbm.at[wid], dst_v)
    plsc.subcore_barrier()

    def body(j, _):
        pltpu.async_copy(u_hbm.at[src_v.at[j]], rows_v, sem).wait()
        pltpu.sync_copy(rows_v, acc.at[dst_v.at[j]], add=True)
        return _
    lax.fori_loop(0, NCH, body, None)

    plsc.subcore_barrier()
    pltpu.sync_copy(
        acc.at[pl.ds(s * RPT, RPT)], out_hbm.at[c, pl.ds(s * RPT, RPT)]
    )


_agg_kernel = functools.partial(
    pl.kernel,
    out_type=jax.ShapeDtypeStruct((NC, NP, D), jnp.float32),
    mesh=_mesh,
    scratch_types=[
        pltpu.VMEM((NCH, CH), jnp.int32),
        pltpu.VMEM((NCH, CH), jnp.int32),
        pltpu.VMEM((CH, D), jnp.float32),
        pltpu.VMEM_SHARED((NP, D), jnp.float32),
        pltpu.SemaphoreType.DMA,
    ],
)(_agg_body)


_BLK = 1000
_GRID = N // _BLK


def _prep_tc_body(degT_ref, x_ref, dinv_ref, u_ref):
    deg = jnp.sum(degT_ref[...], axis=1, keepdims=True) + 1.0
    dinv = lax.rsqrt(deg)
    dinv_ref[...] = dinv
    u_ref[...] = dinv * x_ref[...]


def _prep_tc(degT, x):
    return pl.pallas_call(
        _prep_tc_body,
        grid=(_GRID,),
        in_specs=[
            pl.BlockSpec((_BLK, NC), lambda i: (i, 0)),
            pl.BlockSpec((_BLK, D), lambda i: (i, 0)),
        ],
        out_specs=[
            pl.BlockSpec((_BLK, 1), lambda i: (i, 0)),
            pl.BlockSpec((_BLK, D), lambda i: (i, 0)),
        ],
        out_shape=[
            jax.ShapeDtypeStruct((N, 1), jnp.float32),
            jax.ShapeDtypeStruct((N, D), jnp.float32),
        ],
    )(degT, x)


def _layer_tc_body(p0_ref, p1_ref, u_ref, dinv_ref, scale_ref, w_ref, b_ref,
                   out_ref):
    pre = (p0_ref[...] + p1_ref[...] + u_ref[...]) * dinv_ref[...]
    h = jnp.dot(pre, w_ref[...], preferred_element_type=jnp.float32)
    h = h + b_ref[...]
    out_ref[...] = scale_ref[...] * jnp.maximum(h, 0.0)


def _layer_tc(p0, p1, u, dinv, scale, w, b):
    return pl.pallas_call(
        _layer_tc_body,
        grid=(_GRID,),
        in_specs=[
            pl.BlockSpec((_BLK, D), lambda i: (i, 0)),
            pl.BlockSpec((_BLK, D), lambda i: (i, 0)),
            pl.BlockSpec((_BLK, D), lambda i: (i, 0)),
            pl.BlockSpec((_BLK, 1), lambda i: (i, 0)),
            pl.BlockSpec((_BLK, 1), lambda i: (i, 0)),
            pl.BlockSpec((D, D), lambda i: (0, 0)),
            pl.BlockSpec((1, D), lambda i: (0, 0)),
        ],
        out_specs=pl.BlockSpec((_BLK, D), lambda i: (i, 0)),
        out_shape=jax.ShapeDtypeStruct((N, D), jnp.float32),
    )(p0, p1, u, dinv, scale, w, b)


def kernel(x, edge_index, W1, b1, W2, b2, W3, b3):
    src = edge_index[0]
    dst = edge_index[1]
    pad = EP - E
    src3 = jnp.concatenate(
        [src, jnp.zeros((pad,), jnp.int32)]).reshape(NW, NCH, CH)
    dst3 = jnp.concatenate(
        [dst, jnp.full((pad,), NP - 1, jnp.int32)]).reshape(NW, NCH, CH)

    deg1d = _deg_kernel(dst3)
    degT = deg1d.reshape(NC, NP)[:, :N].T
    dinv2d, u = _prep_tc(degT, x)
    ones2d = jnp.ones((N, 1), jnp.float32)

    for w, b, scale in ((W1, b1, dinv2d), (W2, b2, dinv2d), (W3, b3, ones2d)):
        parts = _agg_kernel(u, src3, dst3)
        u = _layer_tc(parts[0, :N], parts[1, :N], u, dinv2d, scale,
                      w, b.reshape(1, D))
    return u


# speedup vs baseline: 6.3876x; 1.0516x over previous
import functools

import jax
import jax.numpy as jnp
from jax import lax
from jax.experimental import pallas as pl
from jax.experimental.pallas import tpu as pltpu
from jax.experimental.pallas import tpu_sc as plsc

N = 10000
E = 320000
D = 128
NP = 10240

NC = 2
NS = 16
NW = NC * NS
CH = 64
NCH = 160
HCH = 80
EP = NW * NCH * CH
EPW = EP // NW
RPT = NP // NS
ZCH = 128
DZC = NP // 8

_mesh = plsc.VectorSubcoreMesh(
    core_axis_name="c", subcore_axis_name="s", num_cores=NC, num_subcores=NS
)


def _deg_body(dst_hbm, out_hbm, dst_v, ones_v, zeros_v, deg_acc):
    c = lax.axis_index("c")
    s = lax.axis_index("s")
    wid = s * NC + c

    @pl.when(s < 8)
    def _():
        def zfill(i, _):
            zeros_v[pl.ds(i * 16, 16)] = jnp.zeros((16,), jnp.float32)
            return _
        lax.fori_loop(0, DZC // 16, zfill, None)
        pltpu.sync_copy(zeros_v, deg_acc.at[pl.ds(s * DZC, DZC)])

    def ofill(i, _):
        ones_v[pl.ds(i * 16, 16)] = jnp.ones((16,), jnp.float32)
        return _
    lax.fori_loop(0, 128 // 16, ofill, None)

    pltpu.sync_copy(dst_hbm.at[wid], dst_v)
    plsc.subcore_barrier()

    def body(j, _):
        pltpu.sync_copy(
            ones_v.at[pl.ds(0, CH)], deg_acc.at[dst_v.at[j]], add=True
        )
        return _
    lax.fori_loop(0, NCH, body, None)

    plsc.subcore_barrier()

    @pl.when(s == 0)
    def _():
        pltpu.sync_copy(deg_acc, out_hbm.at[pl.ds(c * NP, NP)])


_deg_kernel = functools.partial(
    pl.kernel,
    out_type=jax.ShapeDtypeStruct((NC * NP,), jnp.float32),
    mesh=_mesh,
    scratch_types=[
        pltpu.VMEM((NCH, CH), jnp.int32),
        pltpu.VMEM((128,), jnp.float32),
        pltpu.VMEM((DZC,), jnp.float32),
        pltpu.VMEM_SHARED((NP,), jnp.float32),
    ],
)(_deg_body)


def _agg_body(u_hbm, src_hbm, dst_hbm, out_hbm,
              src_v, dst_v, rows0, rows1, acc, sem0, sem1):
    c = lax.axis_index("c")
    s = lax.axis_index("s")
    wid = s * NC + c

    def zfill16(i, _):
        r = i // (D // 16)
        q = i % (D // 16)
        rows0[r, pl.ds(q * 16, 16)] = jnp.zeros((16,), jnp.float32)
        return _
    lax.fori_loop(0, CH * (D // 16), zfill16, None)

    def zcopy(k, _):
        pltpu.sync_copy(rows0, acc.at[pl.ds(s * RPT + k * CH, CH)])
        return _
    lax.fori_loop(0, RPT // CH, zcopy, None)

    plsc.subcore_barrier()

    for h in range(NCH // HCH):
        pltpu.sync_copy(src_hbm.at[wid, pl.ds(h * HCH, HCH)], src_v)
        pltpu.sync_copy(dst_hbm.at[wid, pl.ds(h * HCH, HCH)], dst_v)
        pltpu.async_copy(u_hbm.at[src_v.at[0]], rows0, sem0)

        def body(k, _):
            j0 = 2 * k
            j1 = 2 * k + 1
            pltpu.make_async_copy(u_hbm.at[src_v.at[j0]], rows0, sem0).wait()
            pltpu.async_copy(u_hbm.at[src_v.at[j1]], rows1, sem1)
            pltpu.sync_copy(rows0, acc.at[dst_v.at[j0]], add=True)
            pltpu.make_async_copy(u_hbm.at[src_v.at[j1]], rows1, sem1).wait()

            @pl.when(k < HCH // 2 - 1)
            def _():
                pltpu.async_copy(u_hbm.at[src_v.at[j0 + 2]], rows0, sem0)

            pltpu.sync_copy(rows1, acc.at[dst_v.at[j1]], add=True)
            return _
        lax.fori_loop(0, HCH // 2, body, None)

    plsc.subcore_barrier()
    pltpu.sync_copy(
        acc.at[pl.ds(s * RPT, RPT)], out_hbm.at[c, pl.ds(s * RPT, RPT)]
    )


_agg_kernel = functools.partial(
    pl.kernel,
    out_type=jax.ShapeDtypeStruct((NC, NP, D), jnp.float32),
    mesh=_mesh,
    scratch_types=[
        pltpu.VMEM((HCH, CH), jnp.int32),
        pltpu.VMEM((HCH, CH), jnp.int32),
        pltpu.VMEM((CH, D), jnp.float32),
        pltpu.VMEM((CH, D), jnp.float32),
        pltpu.VMEM_SHARED((NP, D), jnp.float32),
        pltpu.SemaphoreType.DMA,
        pltpu.SemaphoreType.DMA,
    ],
)(_agg_body)


_BLK = 1000
_GRID = N // _BLK


def _prep_tc_body(degT_ref, x_ref, dinv_ref, u_ref):
    deg = jnp.sum(degT_ref[...], axis=1, keepdims=True) + 1.0
    dinv = lax.rsqrt(deg)
    dinv_ref[...] = dinv
    u_ref[...] = dinv * x_ref[...]


def _prep_tc(degT, x):
    return pl.pallas_call(
        _prep_tc_body,
        grid=(_GRID,),
        in_specs=[
            pl.BlockSpec((_BLK, NC), lambda i: (i, 0)),
            pl.BlockSpec((_BLK, D), lambda i: (i, 0)),
        ],
        out_specs=[
            pl.BlockSpec((_BLK, 1), lambda i: (i, 0)),
            pl.BlockSpec((_BLK, D), lambda i: (i, 0)),
        ],
        out_shape=[
            jax.ShapeDtypeStruct((N, 1), jnp.float32),
            jax.ShapeDtypeStruct((N, D), jnp.float32),
        ],
    )(degT, x)


def _layer_tc_body(p0_ref, p1_ref, u_ref, dinv_ref, scale_ref, w_ref, b_ref,
                   out_ref):
    pre = (p0_ref[...] + p1_ref[...] + u_ref[...]) * dinv_ref[...]
    h = jnp.dot(pre, w_ref[...], preferred_element_type=jnp.float32)
    h = h + b_ref[...]
    out_ref[...] = scale_ref[...] * jnp.maximum(h, 0.0)


def _layer_tc(p0, p1, u, dinv, scale, w, b):
    return pl.pallas_call(
        _layer_tc_body,
        grid=(_GRID,),
        in_specs=[
            pl.BlockSpec((_BLK, D), lambda i: (i, 0)),
            pl.BlockSpec((_BLK, D), lambda i: (i, 0)),
            pl.BlockSpec((_BLK, D), lambda i: (i, 0)),
            pl.BlockSpec((_BLK, 1), lambda i: (i, 0)),
            pl.BlockSpec((_BLK, 1), lambda i: (i, 0)),
            pl.BlockSpec((D, D), lambda i: (0, 0)),
            pl.BlockSpec((1, D), lambda i: (0, 0)),
        ],
        out_specs=pl.BlockSpec((_BLK, D), lambda i: (i, 0)),
        out_shape=jax.ShapeDtypeStruct((N, D), jnp.float32),
    )(p0, p1, u, dinv, scale, w, b)


def kernel(x, edge_index, W1, b1, W2, b2, W3, b3):
    src = edge_index[0]
    dst = edge_index[1]
    pad = EP - E
    src3 = jnp.concatenate(
        [src, jnp.zeros((pad,), jnp.int32)]).reshape(NW, NCH, CH)
    dst3 = jnp.concatenate(
        [dst, jnp.full((pad,), NP - 1, jnp.int32)]).reshape(NW, NCH, CH)

    deg1d = _deg_kernel(dst3)
    degT = deg1d.reshape(NC, NP)[:, :N].T
    dinv2d, u = _prep_tc(degT, x)
    ones2d = jnp.ones((N, 1), jnp.float32)

    for w, b, scale in ((W1, b1, dinv2d), (W2, b2, dinv2d), (W3, b3, ones2d)):
        parts = _agg_kernel(u, src3, dst3)
        u = _layer_tc(parts[0, :N], parts[1, :N], u, dinv2d, scale,
                      w, b.reshape(1, D))
    return u
